# Initial kernel scaffold; baseline (speedup 1.0000x reference)
import jax
import jax.numpy as jnp
from jax.experimental import pallas as pl


def kernel(x, adj, v0_W, v0_b, e0_W, e0_b, v1_W, v1_b, v2_W, v2_b, v3_W, v3_b, v4_W, v4_b, vbn_g, vbn_b, e_W, e_b, ebn_g, ebn_b, e1_W, e1_b):
    raise NotImplementedError("write your pallas kernel here")


# speedup vs baseline: 12.5773x; 12.5773x over previous
import functools

import jax
import jax.numpy as jnp
from jax.experimental import pallas as pl

B, N, D, U, DEPTH = 16, 200, 2, 64, 3
CH = 40
NCH = N // CH
EPS = 1e-5
E_OFF = B * N * (N - 1)


def _diag_mask(c):
    row = jax.lax.broadcasted_iota(jnp.int32, (CH, N), 0)
    col = jax.lax.broadcasted_iota(jnp.int32, (CH, N), 1)
    return col == (c * CH + row)


def _diag_mask3(c):
    row = jax.lax.broadcasted_iota(jnp.int32, (CH, N, U), 0)
    col = jax.lax.broadcasted_iota(jnp.int32, (CH, N, U), 1)
    return col == (c * CH + row)


def _w0_block(adj, e0W, e0b, diag):
    w0 = jnp.maximum(adj[..., None] * e0W[0][None, None, :]
                     + e0b[0][None, None, :], 0.0)
    return jnp.where(diag, 0.0, w0)


def _edge_mm(w, Wmat):
    return jnp.dot(w.reshape(CH * N, U), Wmat,
                   preferred_element_type=jnp.float32).reshape(CH, N, U)


def _stats8(t):
    tf = t.reshape(CH * N, U)
    s = jnp.sum(tf, axis=0, keepdims=True)
    ss = jnp.sum(tf * tf, axis=0, keepdims=True)
    return jnp.concatenate([s, ss, jnp.zeros((6, U), jnp.float32)], axis=0)


def _agg_block(w, x2, diag):
    m = jax.nn.sigmoid(w) * x2[0][None, :, :]
    m = jnp.where(diag, -jnp.inf, m)
    return jnp.max(m, axis=1)


def _passA(adj_ref, e0W_ref, e0b_ref, eW1_ref, c3_ref, x4_ref, x2_ref,
           stats_ref, agg_ref):
    c = pl.program_id(1)
    diag = _diag_mask3(c)
    w0 = _w0_block(adj_ref[0], e0W_ref, e0b_ref, diag)
    t1 = _edge_mm(w0, eW1_ref[...])
    t1 = t1 + c3_ref[0][:, None, :] + x4_ref[0][None, :, :]
    stats_ref[0, 0] = _stats8(t1)
    agg_ref[0] = _agg_block(w0, x2_ref, diag)


def _passB(adj_ref, e0W_ref, e0b_ref, eW1_ref, c31_ref, x41_ref,
           sc1_ref, sh1_ref, eW2_ref, c32_ref, x42_ref, x22_ref,
           w1_ref, stats_ref, agg_ref):
    c = pl.program_id(1)
    diag = _diag_mask3(c)
    w0 = _w0_block(adj_ref[0], e0W_ref, e0b_ref, diag)
    t1 = _edge_mm(w0, eW1_ref[...])
    t1 = t1 + c31_ref[0][:, None, :] + x41_ref[0][None, :, :]
    upd = jnp.maximum(t1 * sc1_ref[0][None, None, :]
                      + sh1_ref[0][None, None, :], 0.0)
    w1 = jnp.where(diag, 0.0, w0 + upd)
    w1_ref[0] = w1
    t2 = _edge_mm(w1, eW2_ref[...])
    t2 = t2 + c32_ref[0][:, None, :] + x42_ref[0][None, :, :]
    stats_ref[0, 0] = _stats8(t2)
    agg_ref[0] = _agg_block(w1, x22_ref, diag)


def _passC(w1_ref, eW2_ref, c32_ref, x42_ref, sc2_ref, sh2_ref,
           eW3_ref, c33_ref, x43_ref, e1W_ref,
           t3_ref, stats_ref, part_ref):
    c = pl.program_id(1)
    diag = _diag_mask3(c)
    w1 = w1_ref[0]
    t2 = _edge_mm(w1, eW2_ref[...])
    t2 = t2 + c32_ref[0][:, None, :] + x42_ref[0][None, :, :]
    upd = jnp.maximum(t2 * sc2_ref[0][None, None, :]
                      + sh2_ref[0][None, None, :], 0.0)
    w2 = jnp.where(diag, 0.0, w1 + upd)
    t3 = _edge_mm(w2, eW3_ref[...])
    t3 = t3 + c33_ref[0][:, None, :] + x43_ref[0][None, :, :]
    t3_ref[0] = t3
    stats_ref[0, 0] = _stats8(t3)
    part_ref[0] = jnp.sum(w2 * e1W_ref[0][None, None, :], axis=-1)


def _passD(t3_ref, part_ref, sc3_ref, sh3_ref, e1W_ref, e1b_ref, out_ref):
    c = pl.program_id(1)
    diag = _diag_mask(c)
    a = jnp.maximum(t3_ref[0] * sc3_ref[0][None, None, :]
                    + sh3_ref[0][None, None, :], 0.0)
    red = jnp.sum(a * e1W_ref[0][None, None, :], axis=-1)
    out = part_ref[0] + red + e1b_ref[0, 0]
    out_ref[0] = jnp.where(diag, 0.0, out)


def _spec_small(shape):
    return pl.BlockSpec(shape, lambda b, c: (0,) * len(shape))


_S_ADJ = pl.BlockSpec((1, CH, N), lambda b, c: (b, c, 0))
_S_NODE_FULL = pl.BlockSpec((1, N, U), lambda b, c: (b, 0, 0))
_S_NODE_CHUNK = pl.BlockSpec((1, CH, U), lambda b, c: (b, c, 0))
_S_BIG = pl.BlockSpec((1, CH, N, U), lambda b, c: (b, c, 0, 0))
_S_STATS = pl.BlockSpec((1, 1, 8, U), lambda b, c: (b, c, 0, 0))
_GRID = (B, NCH)


def _ebn_scale_shift(stats_parts, x3, x4, eb, g, bb):
    s_all = jnp.sum(stats_parts[:, :, 0, :], axis=(0, 1))
    ss_all = jnp.sum(stats_parts[:, :, 1, :], axis=(0, 1))
    td = x3 + x4 + eb[None, :]
    s = s_all - jnp.sum(td, axis=0)
    ss = ss_all - jnp.sum(td * td, axis=0)
    m = s / E_OFF
    v = ss / E_OFF - m * m
    scale = g / jnp.sqrt(v + EPS)
    shift = bb - m * scale
    return scale.reshape(1, U), shift.reshape(1, U)


def _vbn_update(h, x1, agg, g, bb):
    z = x1 + agg
    m = z.mean(axis=0)
    v = z.var(axis=0)
    return h + jnp.maximum((z - m) / jnp.sqrt(v + EPS) * g + bb, 0.0)


@functools.partial(jax.jit, static_argnums=())
def kernel(x, adj, v0_W, v0_b, e0_W, e0_b, v1_W, v1_b, v2_W, v2_b,
           v3_W, v3_b, v4_W, v4_b, vbn_g, vbn_b, e_W, e_b,
           ebn_g, ebn_b, e1_W, e1_b):
    f32 = jnp.float32
    h = jnp.maximum(x.reshape(B * N, D) @ v0_W + v0_b, 0.0)

    def feats(h, i):
        x1 = h @ v1_W[i] + v1_b[i]
        x2 = h @ v2_W[i] + v2_b[i]
        x3 = h @ v3_W[i] + v3_b[i]
        x4 = h @ v4_W[i] + v4_b[i]
        return x1, x2, x3, x4

    def bnu(a):
        return a.reshape(B, N, U)

    e0W2 = e0_W.reshape(1, U)
    e0b2 = e0_b.reshape(1, U)
    e1W2 = e1_W.reshape(U, 1).T.reshape(1, U)
    e1b2 = e1_b.reshape(1, 1)

    x1_1, x2_1, x3_1, x4_1 = feats(h, 0)
    c3_1 = bnu(x3_1 + e_b[0])
    stats1, agg1 = pl.pallas_call(
        _passA,
        grid=_GRID,
        in_specs=[_S_ADJ, _spec_small((1, U)), _spec_small((1, U)),
                  _spec_small((U, U)), _S_NODE_CHUNK, _S_NODE_FULL,
                  _S_NODE_FULL],
        out_specs=[_S_STATS, _S_NODE_CHUNK],
        out_shape=[jax.ShapeDtypeStruct((B, NCH, 8, U), f32),
                   jax.ShapeDtypeStruct((B, N, U), f32)],
    )(adj, e0W2, e0b2, e_W[0], c3_1, bnu(x4_1), bnu(x2_1))
    sc1, sh1 = _ebn_scale_shift(stats1, x3_1, x4_1, e_b[0],
                                ebn_g[0], ebn_b[0])
    h = _vbn_update(h, x1_1, agg1.reshape(B * N, U), vbn_g[0], vbn_b[0])

    x1_2, x2_2, x3_2, x4_2 = feats(h, 1)
    c3_2 = bnu(x3_2 + e_b[1])
    w1, stats2, agg2 = pl.pallas_call(
        _passB,
        grid=_GRID,
        in_specs=[_S_ADJ, _spec_small((1, U)), _spec_small((1, U)),
                  _spec_small((U, U)), _S_NODE_CHUNK, _S_NODE_FULL,
                  _spec_small((1, U)), _spec_small((1, U)),
                  _spec_small((U, U)), _S_NODE_CHUNK, _S_NODE_FULL,
                  _S_NODE_FULL],
        out_specs=[_S_BIG, _S_STATS, _S_NODE_CHUNK],
        out_shape=[jax.ShapeDtypeStruct((B, N, N, U), f32),
                   jax.ShapeDtypeStruct((B, NCH, 8, U), f32),
                   jax.ShapeDtypeStruct((B, N, U), f32)],
    )(adj, e0W2, e0b2, e_W[0], c3_1, bnu(x4_1), sc1, sh1,
      e_W[1], c3_2, bnu(x4_2), bnu(x2_2))
    sc2, sh2 = _ebn_scale_shift(stats2, x3_2, x4_2, e_b[1],
                                ebn_g[1], ebn_b[1])
    h = _vbn_update(h, x1_2, agg2.reshape(B * N, U), vbn_g[1], vbn_b[1])

    _, _, x3_3, x4_3 = feats(h, 2)
    c3_3 = bnu(x3_3 + e_b[2])
    t3, stats3, part = pl.pallas_call(
        _passC,
        grid=_GRID,
        in_specs=[_S_BIG, _spec_small((U, U)), _S_NODE_CHUNK, _S_NODE_FULL,
                  _spec_small((1, U)), _spec_small((1, U)),
                  _spec_small((U, U)), _S_NODE_CHUNK, _S_NODE_FULL,
                  _spec_small((1, U))],
        out_specs=[_S_BIG, _S_STATS, _S_ADJ],
        out_shape=[jax.ShapeDtypeStruct((B, N, N, U), f32),
                   jax.ShapeDtypeStruct((B, NCH, 8, U), f32),
                   jax.ShapeDtypeStruct((B, N, N), f32)],
    )(w1, e_W[1], c3_2, bnu(x4_2), sc2, sh2,
      e_W[2], c3_3, bnu(x4_3), e1W2)
    sc3, sh3 = _ebn_scale_shift(stats3, x3_3, x4_3, e_b[2],
                                ebn_g[2], ebn_b[2])

    out = pl.pallas_call(
        _passD,
        grid=_GRID,
        in_specs=[_S_BIG, _S_ADJ, _spec_small((1, U)), _spec_small((1, U)),
                  _spec_small((1, U)), _spec_small((1, 1))],
        out_specs=_S_ADJ,
        out_shape=jax.ShapeDtypeStruct((B, N, N), f32),
    )(t3, part, sc3, sh3, e1W2, e1b2)
    return out
